# Initial kernel scaffold; baseline (speedup 1.0000x reference)
import jax
import jax.numpy as jnp
from jax.experimental import pallas as pl


def kernel(x, duration, max_len):
    raise NotImplementedError("write your pallas kernel here")


# speedup vs baseline: 6.3499x; 6.3499x over previous
import jax
import jax.numpy as jnp
from jax import lax
from jax.experimental import pallas as pl
from jax.experimental.pallas import tpu as pltpu
from jax.experimental.pallas import tpu_sc as plsc

B, T, D = 16, 1024, 256
L = 8192
NC, NS = 2, 16
NW = NC * NS
PW = B * L // NW
CHUNK = 128
NCHUNK = PW // CHUNK
VL = 16
ZROW = B * T


def _body(x_hbm, dur_hbm, out_hbm, mel_hbm,
          dur_v, a_v, idx_v, buf0, buf1, mel_v,
          gsem0, gsem1, wsem0, wsem1):
    cid = lax.axis_index("c")
    sid = lax.axis_index("s")
    wid = sid * NC + cid
    b = wid // 2
    half = wid % 2
    p0 = half * (L // 2)

    pltpu.sync_copy(dur_hbm.at[b], dur_v.at[pl.ds(0, T)])
    dur_v[pl.ds(T, VL)] = jnp.zeros((VL,), jnp.int32)

    def zero_body(i, _):
        a_v[pl.ds(i * VL, VL)] = jnp.zeros((VL,), jnp.int32)
        return 0
    lax.fori_loop(0, PW // VL, zero_body, 0)

    lane = jnp.arange(VL, dtype=jnp.int32)

    def scat_body(j, carry_base):
        carry, base = carry_base
        v = dur_v[pl.ds(j * VL, VL)]
        s = plsc.cumsum(v) + carry
        i_vec = lane + j * VL
        d_next = dur_v[pl.ds(j * VL + 1, VL)]
        run_end = (d_next != 0) | (i_vec == T - 1)
        local = s - p0
        m = run_end & (local >= 0) & (local < PW)
        plsc.store_scatter(a_v, (jnp.where(m, local, 0),), i_vec + 1, mask=m)
        base = base + jnp.sum(jnp.where(s < p0, 1, 0).astype(jnp.int32))
        return (jnp.max(s), base)

    total, base = lax.fori_loop(
        0, T // VL, scat_body, (jnp.int32(0), jnp.int32(0)))

    rowbase = b * T

    def idx_body(i, carry):
        v = a_v[pl.ds(i * VL, VL)]
        s = jnp.maximum(plsc.cummax(v), carry)
        g = jnp.where(s >= T, ZROW, s + rowbase)
        idx_v[pl.ds(i * VL, VL)] = g
        return jnp.max(s)

    lax.fori_loop(0, PW // VL, idx_body, base)

    @pl.when(half == 0)
    def _():
        mel_v[...] = jnp.full((VL,), total, jnp.int32)
        pltpu.sync_copy(mel_v, mel_hbm.at[b])

    row0 = wid * PW

    def gather(c, buf, gsem):
        return pltpu.async_copy(
            x_hbm.at[idx_v.at[pl.ds(c * CHUNK, CHUNK)]], buf, gsem)

    def write(c, buf, wsem):
        return pltpu.async_copy(
            buf, out_hbm.at[pl.ds(row0 + c * CHUNK, CHUNK)], wsem)

    gather(0, buf0, gsem0).wait()

    def pipe_body(cc, _):
        c = cc * 2
        g1 = gather(c + 1, buf1, gsem1)
        w0 = write(c, buf0, wsem0)
        g1.wait()
        w0.wait()

        @pl.when(c + 2 < NCHUNK)
        def _():
            g0 = gather(c + 2, buf0, gsem0)
            w1 = write(c + 1, buf1, wsem1)
            g0.wait()
            w1.wait()

        @pl.when(c + 2 >= NCHUNK)
        def _():
            write(c + 1, buf1, wsem1).wait()
        return 0

    lax.fori_loop(0, NCHUNK // 2, pipe_body, 0)


import functools


@functools.cache
def _regulate():
    return pl.kernel(
        _body,
        out_type=[
            jax.ShapeDtypeStruct((B * L, D), jnp.float32),
            jax.ShapeDtypeStruct((B, VL), jnp.int32),
        ],
        mesh=plsc.VectorSubcoreMesh(core_axis_name="c", subcore_axis_name="s",
                                    num_cores=NC, num_subcores=NS),
        compiler_params=pltpu.CompilerParams(needs_layout_passes=False),
        scratch_types=[
            pltpu.VMEM((T + VL,), jnp.int32),
            pltpu.VMEM((PW,), jnp.int32),
            pltpu.VMEM((PW,), jnp.int32),
            pltpu.VMEM((CHUNK, D), jnp.float32),
            pltpu.VMEM((CHUNK, D), jnp.float32),
            pltpu.VMEM((VL,), jnp.int32),
            pltpu.SemaphoreType.DMA,
            pltpu.SemaphoreType.DMA,
            pltpu.SemaphoreType.DMA,
            pltpu.SemaphoreType.DMA,
        ],
    )


def kernel(x, duration, max_len):
    x_pad = jnp.concatenate(
        [x.reshape(B * T, D), jnp.zeros((8, D), x.dtype)], axis=0)
    out_flat, mel2 = _regulate()(x_pad, duration.astype(jnp.int32))
    return out_flat.reshape(B, L, D), mel2[:, 0]
